# baseline (speedup 1.0000x reference)
import functools

import jax
import jax.numpy as jnp
import numpy as np
from jax import lax
from jax.experimental import pallas as pl
from jax.experimental.pallas import tpu as pltpu
from jax.experimental.pallas import tpu_sc as plsc

N = 10000
E = 320000
D = 128
OUT = 128

NUM_CORES = 2
NUM_SUBCORES = 16
NUM_WORKERS = NUM_CORES * NUM_SUBCORES
CHUNK = 128
IDX_CH = 16
N_STAGES = 5
N_CHUNKS = N_STAGES * IDX_CH
E_PER_W = CHUNK * N_CHUNKS
N_REAL_CHUNKS = E // CHUNK
N_PAD = 10240
ROWS_PER_TILE = N_PAD // NUM_SUBCORES
LAST_W = NUM_WORKERS - 1
W31_REAL = N_REAL_CHUNKS - LAST_W * N_CHUNKS

_W31_PAD = np.stack([
    np.arange((N_CHUNKS - W31_REAL) * CHUNK) % N,
    N + np.arange((N_CHUNKS - W31_REAL) * CHUNK) % (N_PAD - N),
]).astype(np.int32).reshape(2, N_CHUNKS - W31_REAL, CHUNK)

_ZEROS = np.zeros((ROWS_PER_TILE, D), np.float32)


def _sc_aggregate(features, ei, w31, zeros):
    mesh = plsc.VectorSubcoreMesh(core_axis_name="c", subcore_axis_name="s")

    @functools.partial(
        pl.kernel,
        mesh=mesh,
        out_type=jax.ShapeDtypeStruct((NUM_CORES, N_PAD, D), jnp.float32),
        scratch_types=[
            pltpu.VMEM((2, 2, IDX_CH, CHUNK), jnp.int32),
            pltpu.VMEM((2, CHUNK, D), jnp.float32),
            pltpu.VMEM_SHARED((N_PAD, D), jnp.float32),
            pltpu.SemaphoreType.DMA,
            pltpu.SemaphoreType.DMA,
            pltpu.SemaphoreType.DMA,
            pltpu.SemaphoreType.DMA,
        ],
    )
    def agg(features_hbm, ei_hbm, w31_hbm, zeros_hbm, out_hbm,
            idx_v, rows_v, acc_sh, sem_ga, sem_gb, sem_i, sem_z):
        rows_a = rows_v.at[0]
        rows_b = rows_v.at[1]
        c = lax.axis_index("c")
        s = lax.axis_index("s")
        wid = c * NUM_SUBCORES + s

        def start_gather(j, buf, sem):
            src_row = idx_v.at[(j // IDX_CH) % 2, 0, j % IDX_CH]
            pltpu.make_async_copy(features_hbm.at[src_row], buf, sem).start()

        def wait_dma(buf, sem):
            pltpu.make_async_copy(features_hbm.at[idx_v.at[0, 0, 0]], buf, sem).wait()

        def scatter_add(j, buf):
            dst_row = idx_v.at[(j // IDX_CH) % 2, 1, j % IDX_CH]
            pltpu.sync_copy(buf, acc_sh.at[dst_row], add=True)

        def start_idx_load(stage, buf):
            @pl.when(wid < LAST_W)
            def _():
                base = wid * N_CHUNKS + stage * IDX_CH
                pltpu.make_async_copy(ei_hbm.at[0, pl.ds(base, IDX_CH)], idx_v.at[buf, 0], sem_i).start()
                pltpu.make_async_copy(ei_hbm.at[1, pl.ds(base, IDX_CH)], idx_v.at[buf, 1], sem_i).start()

            @pl.when(wid == LAST_W)
            def _():
                pltpu.make_async_copy(w31_hbm.at[0, stage], idx_v.at[buf, 0], sem_i).start()
                pltpu.make_async_copy(w31_hbm.at[1, stage], idx_v.at[buf, 1], sem_i).start()

        def wait_idx_load():
            pltpu.make_async_copy(w31_hbm.at[0, 0], idx_v.at[0], sem_i).wait()

        zero_slice = acc_sh.at[pl.ds(s * ROWS_PER_TILE, ROWS_PER_TILE)]
        pltpu.make_async_copy(zeros_hbm, zero_slice, sem_z).start()
        @pl.when(wid < LAST_W)
        def _():
            pltpu.sync_copy(ei_hbm.at[0, pl.ds(wid * N_CHUNKS, IDX_CH)], idx_v.at[0, 0])
            pltpu.sync_copy(ei_hbm.at[1, pl.ds(wid * N_CHUNKS, IDX_CH)], idx_v.at[0, 1])

        @pl.when(wid == LAST_W)
        def _():
            pltpu.sync_copy(w31_hbm.at[0, 0], idx_v.at[0, 0])
            pltpu.sync_copy(w31_hbm.at[1, 0], idx_v.at[0, 1])

        start_idx_load(1, 1)
        start_gather(0, rows_a, sem_ga)
        start_gather(1, rows_b, sem_gb)
        pltpu.make_async_copy(zeros_hbm, zero_slice, sem_z).wait()
        plsc.subcore_barrier()

        def body(i, carry):
            j0 = 2 * i
            stg = j0 // IDX_CH

            @pl.when(j0 % IDX_CH == IDX_CH - 2)
            def _():
                wait_idx_load()

            wait_dma(rows_a, sem_ga)
            scatter_add(j0, rows_a)
            start_gather(j0 + 2, rows_a, sem_ga)
            wait_dma(rows_b, sem_gb)
            scatter_add(j0 + 1, rows_b)

            @pl.when((j0 % IDX_CH == IDX_CH - 2) & (stg + 2 < N_STAGES))
            def _():
                start_idx_load(stg + 2, stg % 2)

            start_gather(j0 + 3, rows_b, sem_gb)
            return carry

        lax.fori_loop(0, N_CHUNKS // 2 - 1, body, 0)
        wait_dma(rows_a, sem_ga)
        scatter_add(N_CHUNKS - 2, rows_a)
        wait_dma(rows_b, sem_gb)
        scatter_add(N_CHUNKS - 1, rows_b)
        plsc.subcore_barrier()
        pltpu.sync_copy(
            acc_sh.at[pl.ds(s * ROWS_PER_TILE, ROWS_PER_TILE)],
            out_hbm.at[c, pl.ds(s * ROWS_PER_TILE, ROWS_PER_TILE)],
        )

    return agg(features, ei, w31, zeros)


def _tc_kernel(p_ref, w_ref, b_ref, o_ref):
    acc = p_ref[0] + p_ref[1]
    y = jnp.dot(acc, w_ref[...], preferred_element_type=jnp.float32)
    o_ref[...] = jnp.maximum(y + b_ref[...], 0.0)


def _tc_transform(partials, W, b):
    bn = 1000
    grid = (N // bn,)
    return pl.pallas_call(
        _tc_kernel,
        grid=grid,
        in_specs=[
            pl.BlockSpec((NUM_CORES, bn, D), lambda i: (0, i, 0)),
            pl.BlockSpec((D, OUT), lambda i: (0, 0)),
            pl.BlockSpec((1, OUT), lambda i: (0, 0)),
        ],
        out_specs=pl.BlockSpec((bn, OUT), lambda i: (i, 0)),
        out_shape=jax.ShapeDtypeStruct((N, OUT), jnp.float32),
    )(partials, W, b)


def kernel(features, edge_index, W, b):
    ei = edge_index.astype(jnp.int32)
    w31 = jnp.concatenate(
        [ei[:, LAST_W * E_PER_W:].reshape(2, W31_REAL, CHUNK), jnp.asarray(_W31_PAD)],
        axis=1,
    ).reshape(2, N_STAGES, IDX_CH, CHUNK)
    ei = ei.reshape(2, N_REAL_CHUNKS, CHUNK)
    zeros = jnp.asarray(_ZEROS)
    partials = _sc_aggregate(features, ei, w31, zeros)
    return _tc_transform(partials, W, b)


# speedup vs baseline: 1.0313x; 1.0313x over previous
import functools

import jax
import jax.numpy as jnp
import numpy as np
from jax import lax
from jax.experimental import pallas as pl
from jax.experimental.pallas import tpu as pltpu
from jax.experimental.pallas import tpu_sc as plsc

N = 10000
E = 320000
D = 128
OUT = 128

NUM_CORES = 2
NUM_SUBCORES = 16
NUM_WORKERS = NUM_CORES * NUM_SUBCORES
CHUNK = 128
IDX_CH = 16
N_STAGES = 5
N_CHUNKS = N_STAGES * IDX_CH
E_PER_W = CHUNK * N_CHUNKS
N_REAL_CHUNKS = E // CHUNK
N_PAD = 10240
ROWS_PER_TILE = N_PAD // NUM_SUBCORES
LAST_W = NUM_WORKERS - 1
W31_REAL = N_REAL_CHUNKS - LAST_W * N_CHUNKS

_W31_PAD = np.stack([
    np.arange((N_CHUNKS - W31_REAL) * CHUNK) % N,
    N + np.arange((N_CHUNKS - W31_REAL) * CHUNK) % (N_PAD - N),
]).astype(np.int32).reshape(2, N_CHUNKS - W31_REAL, CHUNK)

_ZEROS = np.zeros((ROWS_PER_TILE, D), np.float32)


def _sc_aggregate(features, ei, w31, zeros):
    mesh = plsc.VectorSubcoreMesh(core_axis_name="c", subcore_axis_name="s")

    @functools.partial(
        pl.kernel,
        mesh=mesh,
        out_type=jax.ShapeDtypeStruct((NUM_CORES, N_PAD, D), jnp.float32),
        scratch_types=[
            pltpu.VMEM((2, 2, IDX_CH, CHUNK), jnp.int32),
            pltpu.VMEM((2, CHUNK, D), jnp.float32),
            pltpu.VMEM_SHARED((N_PAD, D), jnp.float32),
            pltpu.SemaphoreType.DMA,
            pltpu.SemaphoreType.DMA,
            pltpu.SemaphoreType.DMA,
            pltpu.SemaphoreType.DMA,
        ],
    )
    def agg(features_hbm, ei_hbm, w31_hbm, zeros_hbm, out_hbm,
            idx_v, rows_v, acc_sh, sem_ga, sem_gb, sem_i, sem_z):
        rows_a = rows_v.at[0]
        rows_b = rows_v.at[1]
        c = lax.axis_index("c")
        s = lax.axis_index("s")
        wid = c * NUM_SUBCORES + s

        def start_gather(j, buf, sem):
            src_row = idx_v.at[(j // IDX_CH) % 2, 0, j % IDX_CH]
            pltpu.make_async_copy(features_hbm.at[src_row], buf, sem).start()

        def wait_dma(buf, sem):
            pltpu.make_async_copy(features_hbm.at[idx_v.at[0, 0, 0]], buf, sem).wait()

        def scatter_add(j, buf):
            dst_row = idx_v.at[(j // IDX_CH) % 2, 1, j % IDX_CH]
            pltpu.sync_copy(buf, acc_sh.at[dst_row], add=True)

        def start_idx_load(stage, buf):
            @pl.when(wid < LAST_W)
            def _():
                base = wid * N_CHUNKS + stage * IDX_CH
                pltpu.make_async_copy(ei_hbm.at[0, pl.ds(base, IDX_CH)], idx_v.at[buf, 0], sem_i).start()
                pltpu.make_async_copy(ei_hbm.at[1, pl.ds(base, IDX_CH)], idx_v.at[buf, 1], sem_i).start()

            @pl.when(wid == LAST_W)
            def _():
                pltpu.make_async_copy(w31_hbm.at[0, stage], idx_v.at[buf, 0], sem_i).start()
                pltpu.make_async_copy(w31_hbm.at[1, stage], idx_v.at[buf, 1], sem_i).start()

        def wait_idx_load():
            pltpu.make_async_copy(w31_hbm.at[0, 0], idx_v.at[0], sem_i).wait()

        zero_slice = acc_sh.at[pl.ds(s * ROWS_PER_TILE, ROWS_PER_TILE)]
        pltpu.make_async_copy(zeros_hbm, zero_slice, sem_z).start()
        @pl.when(wid < LAST_W)
        def _():
            pltpu.sync_copy(ei_hbm.at[0, pl.ds(wid * N_CHUNKS, IDX_CH)], idx_v.at[0, 0])
            pltpu.sync_copy(ei_hbm.at[1, pl.ds(wid * N_CHUNKS, IDX_CH)], idx_v.at[0, 1])

        @pl.when(wid == LAST_W)
        def _():
            pltpu.sync_copy(w31_hbm.at[0, 0], idx_v.at[0, 0])
            pltpu.sync_copy(w31_hbm.at[1, 0], idx_v.at[0, 1])

        start_idx_load(1, 1)
        start_gather(0, rows_a, sem_ga)
        start_gather(1, rows_b, sem_gb)
        pltpu.make_async_copy(zeros_hbm, zero_slice, sem_z).wait()
        plsc.subcore_barrier()

        def body(i, carry):
            j0 = 2 * i
            stg = j0 // IDX_CH

            @pl.when(j0 % IDX_CH == IDX_CH - 2)
            def _():
                wait_idx_load()

            wait_dma(rows_a, sem_ga)
            scatter_add(j0, rows_a)
            start_gather(j0 + 2, rows_a, sem_ga)
            wait_dma(rows_b, sem_gb)
            scatter_add(j0 + 1, rows_b)

            @pl.when((j0 % IDX_CH == IDX_CH - 2) & (stg + 2 < N_STAGES))
            def _():
                start_idx_load(stg + 2, stg % 2)

            start_gather(j0 + 3, rows_b, sem_gb)
            return carry

        lax.fori_loop(0, N_CHUNKS // 2 - 1, body, 0)
        wait_dma(rows_a, sem_ga)
        scatter_add(N_CHUNKS - 2, rows_a)
        wait_dma(rows_b, sem_gb)
        scatter_add(N_CHUNKS - 1, rows_b)
        plsc.subcore_barrier()
        pltpu.sync_copy(
            acc_sh.at[pl.ds(s * ROWS_PER_TILE, ROWS_PER_TILE)],
            out_hbm.at[c, pl.ds(s * ROWS_PER_TILE, ROWS_PER_TILE)],
        )

    return agg(features, ei, w31, zeros)


def _tc_kernel(p_ref, w_ref, b_ref, o_ref):
    acc = p_ref[0] + p_ref[1]
    y = jnp.dot(acc, w_ref[...], preferred_element_type=jnp.float32)
    o_ref[...] = jnp.maximum(y + b_ref[...], 0.0)


def _tc_transform(partials, W, b):
    bn = 5000
    grid = (N // bn,)
    return pl.pallas_call(
        _tc_kernel,
        grid=grid,
        in_specs=[
            pl.BlockSpec((NUM_CORES, bn, D), lambda i: (0, i, 0)),
            pl.BlockSpec((D, OUT), lambda i: (0, 0)),
            pl.BlockSpec((1, OUT), lambda i: (0, 0)),
        ],
        out_specs=pl.BlockSpec((bn, OUT), lambda i: (i, 0)),
        out_shape=jax.ShapeDtypeStruct((N, OUT), jnp.float32),
    )(partials, W, b)


def kernel(features, edge_index, W, b):
    ei = edge_index.astype(jnp.int32)
    w31 = jnp.concatenate(
        [ei[:, LAST_W * E_PER_W:].reshape(2, W31_REAL, CHUNK), jnp.asarray(_W31_PAD)],
        axis=1,
    ).reshape(2, N_STAGES, IDX_CH, CHUNK)
    ei = ei.reshape(2, N_REAL_CHUNKS, CHUNK)
    zeros = jnp.asarray(_ZEROS)
    partials = _sc_aggregate(features, ei, w31, zeros)
    return _tc_transform(partials, W, b)
